# baseline (speedup 1.0000x reference)
import functools

import jax
import jax.numpy as jnp
from jax import lax
from jax.experimental import pallas as pl
from jax.experimental.pallas import tpu as pltpu
from jax.experimental.pallas import tpu_sc as plsc


def kernel(inputs, table):
    B, S = inputs.shape
    V, D = table.shape
    idx_p = jnp.pad(inputs, ((0, 0), (0, 256 - S)))
    table2 = table.reshape(V // 2, 2 * D)

    info = plsc.get_sparse_core_info()
    NC, NS = info.num_cores, info.num_subcores
    NW = NC * NS
    slabs_per_w = B // NW
    SP = S // 2

    mesh = plsc.VectorSubcoreMesh(core_axis_name="c", subcore_axis_name="s")

    @functools.partial(
        pl.kernel,
        mesh=mesh,
        out_type=jax.ShapeDtypeStruct((B * S // 2, 2 * D), jnp.float32),
        scratch_types=[
            pltpu.VMEM((slabs_per_w, 256), jnp.int32),
            pltpu.VMEM((2, 2 * SP, 2 * D), jnp.float32),
            pltpu.SemaphoreType.DMA((2,)),
            pltpu.SemaphoreType.DMA((2,)),
        ],
    )
    def gather_kernel(idx_hbm, table_hbm, out_hbm, idx_v, rows_c,
                      sem_g, sem_w):
        wid = lax.axis_index("s") * NC + lax.axis_index("c")
        slab0 = wid * slabs_per_w
        prow0 = slab0 * SP

        pltpu.sync_copy(idx_hbm.at[pl.ds(slab0, slabs_per_w), :], idx_v)

        def fire_one(q, j_half, k, s):
            r = s >> 1
            h0 = (s & 1) == 0
            dst_off = (k & 1) * D

            @pl.when(h0)
            def _():
                pltpu.async_copy(
                    table_hbm.at[r, pl.ds(0, D)],
                    rows_c.at[q, j_half, pl.ds(dst_off, D)],
                    sem_g.at[q],
                )

            @pl.when(jnp.logical_not(h0))
            def _():
                pltpu.async_copy(
                    table_hbm.at[r, pl.ds(D, D)],
                    rows_c.at[q, j_half, pl.ds(dst_off, D)],
                    sem_g.at[q],
                )

        def fire_rows(i, q, hs):
            base = hs * SP

            def grp(g, carry):
                vec = idx_v[i, pl.ds(g * 16, 16)]
                for k in range(16):
                    fire_one(q, base + g * 8 + k // 2, k, vec[k])
                return carry

            lax.fori_loop(0, S // 16, grp, 0)
            g_t = S // 16
            vec = idx_v[i, pl.ds(g_t * 16, 16)]
            for k in range(S - g_t * 16):
                fire_one(q, base + g_t * 8 + k // 2, k, vec[k])

        def fire_pair(ii, q):
            fire_rows(2 * ii, q, 0)
            fire_rows(2 * ii + 1, q, 1)

        def drain_rows(q):
            def row(j, carry):
                pltpu.make_async_copy(
                    table_hbm.at[0, pl.ds(0, D)],
                    rows_c.at[q, 0, pl.ds(0, D)],
                    sem_g.at[q],
                ).wait()
                return carry

            lax.fori_loop(0, 2 * S, row, 0)

        def wait_write(q):
            pltpu.make_async_copy(
                rows_c.at[q], out_hbm.at[pl.ds(prow0, 2 * SP)], sem_w.at[q]
            ).wait()

        n_pairs = slabs_per_w // 2
        fire_pair(0, 0)

        def body(g, carry):
            for p in (0, 1):
                ii = 2 * g + p
                np_ = 1 - p

                @pl.when(ii >= 1)
                def _():
                    wait_write(np_)

                @pl.when(ii + 1 < n_pairs)
                def _():
                    fire_pair(ii + 1, np_)

                drain_rows(p)
                pltpu.async_copy(
                    rows_c.at[p],
                    out_hbm.at[pl.ds(prow0 + ii * 2 * SP, 2 * SP)],
                    sem_w.at[p],
                )
            return carry

        lax.fori_loop(0, n_pairs // 2, body, 0)
        wait_write(1)

    out_x = gather_kernel(idx_p, table2)
    return out_x.reshape(B, S, D)


# speedup vs baseline: 1.3629x; 1.3629x over previous
import functools

import jax
import jax.numpy as jnp
from jax import lax
from jax.experimental import pallas as pl
from jax.experimental.pallas import tpu as pltpu
from jax.experimental.pallas import tpu_sc as plsc


def kernel(inputs, table):
    B, S = inputs.shape
    V, D = table.shape
    idx_p = jnp.pad(inputs, ((0, 0), (0, 256 - S)))

    info = plsc.get_sparse_core_info()
    NC, NS = info.num_cores, info.num_subcores
    NW = NC * NS
    slabs_per_w = B // NW
    n_pairs = slabs_per_w // 2

    mesh = plsc.VectorSubcoreMesh(core_axis_name="c", subcore_axis_name="s")

    @functools.partial(
        pl.kernel,
        mesh=mesh,
        out_type=jax.ShapeDtypeStruct((B, S, D), jnp.float32),
        scratch_types=[
            pltpu.VMEM((slabs_per_w // 2, 256), jnp.int32),
            pltpu.VMEM((2, 2 * S, D), jnp.float32),
            pltpu.SemaphoreType.DMA((2,)),
            pltpu.SemaphoreType.DMA((2,)),
        ],
    )
    def gather_kernel(idx_hbm, table_hbm, out_hbm, idx_v, rows_c,
                      sem_g, sem_w):
        wid = lax.axis_index("s") * NC + lax.axis_index("c")
        slab0 = wid * slabs_per_w

        pltpu.sync_copy(idx_hbm.at[pl.ds(slab0, slabs_per_w // 2), :], idx_v)

        def fire_rows(i, q, hs):
            def grp(g, carry):
                vec = idx_v[i & (slabs_per_w // 2 - 1), pl.ds(g * 16, 16)]
                for k in range(16):
                    pltpu.async_copy(
                        table_hbm.at[vec[k]],
                        rows_c.at[q, hs * S + g * 16 + k],
                        sem_g.at[q],
                    )
                return carry

            lax.fori_loop(0, S // 16, grp, 0)
            g_t = S // 16
            vec = idx_v[i & (slabs_per_w // 2 - 1), pl.ds(g_t * 16, 16)]
            for k in range(S - g_t * 16):
                pltpu.async_copy(
                    table_hbm.at[vec[k]],
                    rows_c.at[q, hs * S + g_t * 16 + k],
                    sem_g.at[q],
                )

        def fire_pair(ii, q):
            fire_rows(2 * ii, q, 0)
            fire_rows(2 * ii + 1, q, 1)

        def drain_rows(q):
            def row(j, carry):
                pltpu.make_async_copy(
                    table_hbm.at[0], rows_c.at[q, 0], sem_g.at[q]
                ).wait()
                return carry

            lax.fori_loop(0, 2 * S, row, 0)

        def wait_write(q):
            pltpu.make_async_copy(
                rows_c.at[q],
                out_hbm.at[pl.ds(slab0, 2)].reshape(2 * S, D),
                sem_w.at[q],
            ).wait()

        fire_pair(0, 0)

        def body(g, carry):
            for p in (0, 1):
                ii = 2 * g + p
                np_ = 1 - p

                @pl.when(ii >= 1)
                def _():
                    wait_write(np_)

                @pl.when(2 * (ii + 1) == slabs_per_w // 2)
                def _():
                    pltpu.sync_copy(
                        idx_hbm.at[pl.ds(slab0 + slabs_per_w // 2,
                                         slabs_per_w // 2), :],
                        idx_v,
                    )

                @pl.when(ii + 1 < n_pairs)
                def _():
                    fire_pair(ii + 1, np_)

                drain_rows(p)
                pltpu.async_copy(
                    rows_c.at[p],
                    out_hbm.at[pl.ds(slab0 + 2 * ii, 2)].reshape(2 * S, D),
                    sem_w.at[p],
                )
            return carry

        lax.fori_loop(0, n_pairs // 2, body, 0)
        wait_write(1)

    return gather_kernel(idx_p, table)
